# baseline (speedup 1.0000x reference)
import jax
import jax.numpy as jnp
from jax.experimental import pallas as pl
from jax.experimental.pallas import tpu as pltpu

_NUM_CLASSES = 9605
_BATCH = 1024
_CLIP = 0.05
_EPS = 1e-08
_ALPHA3 = 2.0
_TOPK = 10
_ROWS_PER_BLOCK = 64
_LANES = 128
_FULL_CHUNKS = _NUM_CLASSES // _LANES
_NEG = -3e38


def _loss_kernel(x_ref, y_ref, out_ref):
    x = x_ref[...]
    yi = y_ref[...]
    pos = yi == 1

    xs = jax.nn.sigmoid(x)
    xs_neg = jnp.minimum((1.0 - xs) + _CLIP, 1.0)
    pt = jnp.where(pos, xs, xs_neg)
    loss = jnp.log(jnp.maximum(pt, _EPS))
    one_m_pt = 1.0 - pt
    w = jnp.where(pos, one_m_pt, (one_m_pt * one_m_pt) * (one_m_pt * one_m_pt))
    partial = jnp.sum(loss * w)

    has_c = jnp.sum(yi[:, 0:30], axis=1) > 0
    has_r = jnp.sum(yi[:, 100:170], axis=1) > 0
    has_d = jnp.sum(yi[:, 300:370], axis=1) > 0
    gt_none = jnp.logical_not(has_c | has_r | has_d)

    key = jax.lax.bitcast_convert_type(
        (jax.lax.bitcast_convert_type(x, jnp.int32) & jnp.int32(-2)) | yi,
        jnp.float32,
    )

    shp = (x.shape[0], _LANES)
    lane = jax.lax.broadcasted_iota(jnp.int32, shp, 1)
    b1 = jnp.full(shp, _NEG, jnp.float32)
    b2 = jnp.full(shp, _NEG, jnp.float32)
    i1 = jnp.zeros(shp, jnp.int32)
    i2 = jnp.zeros(shp, jnp.int32)
    for c in range(_FULL_CHUNKS + 1):
        if c < _FULL_CHUNKS:
            v = key[:, c * _LANES:(c + 1) * _LANES]
            idx = lane + jnp.int32(c * _LANES)
        else:
            v = key[:, _NUM_CLASSES - _LANES:_NUM_CLASSES]
            v = jnp.where(lane < (_LANES - _NUM_CLASSES % _LANES), _NEG, v)
            idx = lane + jnp.int32(_NUM_CLASSES - _LANES)
        gt1 = v > b1
        gt2 = v > b2
        b2 = jnp.where(gt1, b1, jnp.where(gt2, v, b2))
        i2 = jnp.where(gt1, i1, jnp.where(gt2, idx, i2))
        b1 = jnp.where(gt1, v, b1)
        i1 = jnp.where(gt1, idx, i1)

    cand = jnp.concatenate([b1, b2], axis=1)
    cidx = jnp.concatenate([i1, i2], axis=1)

    corr = jnp.float32(0.0)
    for _ in range(_TOPK):
        m = jnp.max(cand, axis=1)
        eq = cand == m[:, None]
        sel = jnp.min(jnp.where(eq, cidx, jnp.int32(2**30)), axis=1)
        onehot = eq & (cidx == sel[:, None])
        cand = jnp.where(onehot, _NEG, cand)

        kb = jax.lax.bitcast_convert_type(m, jnp.int32)
        yj = (kb & 1).astype(jnp.float32)
        xs_j = jax.nn.sigmoid(m)
        xsn_j = jnp.minimum((1.0 - xs_j) + _CLIP, 1.0)
        loss_j = jnp.where(
            yj == 1.0,
            jnp.log(jnp.maximum(xs_j, _EPS)),
            jnp.log(jnp.maximum(xsn_j, _EPS)),
        )
        ompj = jnp.where(yj == 1.0, 1.0 - xs_j, 1.0 - xsn_j)
        wj = jnp.where(yj == 1.0, ompj, (ompj * ompj) * (ompj * ompj))
        lwj = loss_j * wj
        factor = jnp.where(yj == 0.0, xs_j, xsn_j) * _ALPHA3

        is_c = sel < 30
        is_r = (sel >= 100) & (sel < 170)
        is_d = (sel >= 300) & (sel < 370)
        is_4 = jnp.logical_not(is_c | is_r | is_d)
        cond = (is_c & has_c) | (is_r & has_r) | (is_d & has_d) | (is_4 & gt_none)
        corr = corr + jnp.sum(jnp.where(cond, lwj * (factor - 1.0), 0.0))

    @pl.when(pl.program_id(0) == 0)
    def _():
        out_ref[...] = jnp.zeros_like(out_ref)

    out_ref[...] += jnp.reshape(partial + corr, (1, 1))


@jax.jit
def kernel(x, y):
    grid = _BATCH // _ROWS_PER_BLOCK
    out = pl.pallas_call(
        _loss_kernel,
        grid=(grid,),
        in_specs=[
            pl.BlockSpec((_ROWS_PER_BLOCK, _NUM_CLASSES), lambda i: (i, 0)),
            pl.BlockSpec((_ROWS_PER_BLOCK, _NUM_CLASSES), lambda i: (i, 0)),
        ],
        out_specs=pl.BlockSpec((1, 1), lambda i: (0, 0)),
        out_shape=jax.ShapeDtypeStruct((1, 1), jnp.float32),
        compiler_params=pltpu.CompilerParams(dimension_semantics=("arbitrary",)),
    )(x, y)
    return -out[0, 0]
